# Initial kernel scaffold; baseline (speedup 1.0000x reference)
import jax
import jax.numpy as jnp
from jax.experimental import pallas as pl


def kernel(x, position_weight, level_weight, centroid_weight):
    raise NotImplementedError("write your pallas kernel here")


# speedup vs baseline: 4.5692x; 4.5692x over previous
import functools
import jax
import jax.numpy as jnp
from jax.experimental import pallas as pl
from jax.experimental.pallas import tpu as pltpu

_B, _SIZE = 128, 28
_P = _SIZE * _SIZE
_P2 = 896
_D = 2048
_L = 256
_C = 10


def _encode_body(x_ref, pos_ref, lev_ref, cent_ref, out_ref):
    L = _L
    xr = x_ref[0]
    idx = jnp.clip(jnp.round(xr * (L - 1)), 0, L - 1).astype(jnp.int32)
    lvl_iota = jax.lax.broadcasted_iota(jnp.int32, (L, _P2), 0)
    onehot = (lvl_iota == idx).astype(jnp.bfloat16)
    m = jnp.dot(onehot, pos_ref[...],
                preferred_element_type=jnp.float32)
    s = jnp.sum(lev_ref[...].astype(jnp.float32) * m, axis=0, keepdims=True)
    y = jnp.where(s > 0.0, 1.0, -1.0).astype(jnp.float32)
    cent = cent_ref[...]
    wn = cent / (jnp.sqrt(jnp.sum(cent * cent, axis=0, keepdims=True)) + 1e-12)
    yn = y / jnp.sqrt(jnp.float32(_D))
    out_ref[0] = jnp.dot(yn, wn, preferred_element_type=jnp.float32)


@jax.jit
def _run(x, position_weight, level_weight, centroid_weight):
    b = x.shape[0]
    flat = x.reshape(b, _P)
    flat = jnp.pad(flat, ((0, 0), (0, _P2 - _P)))
    x3 = flat.reshape(b, 1, _P2)
    pos = jnp.pad(position_weight.astype(jnp.bfloat16),
                  ((0, _P2 - _P), (0, 0)))
    lev = level_weight.astype(jnp.bfloat16)
    cent_t = centroid_weight.T

    out = pl.pallas_call(
        _encode_body,
        grid=(b,),
        in_specs=[
            pl.BlockSpec((1, 1, _P2), lambda i: (i, 0, 0)),
            pl.BlockSpec((_P2, _D), lambda i: (0, 0)),
            pl.BlockSpec((_L, _D), lambda i: (0, 0)),
            pl.BlockSpec((_D, _C), lambda i: (0, 0)),
        ],
        out_specs=pl.BlockSpec((1, 1, _C), lambda i: (i, 0, 0)),
        out_shape=jax.ShapeDtypeStruct((b, 1, _C), jnp.float32),
        compiler_params=pltpu.CompilerParams(
            dimension_semantics=("arbitrary",)),
    )(x3, pos, lev, cent_t)
    return out.reshape(b, _C)


def kernel(x, position_weight, level_weight, centroid_weight):
    return _run(x, position_weight, level_weight, centroid_weight)
